# baseline (speedup 1.0000x reference)
import functools

import jax
import jax.numpy as jnp
from jax import lax
from jax.experimental import pallas as pl
from jax.experimental.pallas import tpu as pltpu
from jax.experimental.pallas import tpu_sc as plsc

_V = 100000
_D = 128
_L = 200
_K = 64
_N = 16
_B_OBS = _L * _N
_B_CAND = _L * _K
_B_TOTAL = _B_OBS + _B_CAND + _B_OBS

_NC = 2
_NS = 16
_B_PER_W = 640
_NACT = _B_TOTAL // _B_PER_W
_CHUNK = 80
_NCHUNK = _B_PER_W // _CHUNK


def _sc_gather(table, obs_idx, cand_idx, hid_idx):
  mesh = plsc.VectorSubcoreMesh(core_axis_name="c", subcore_axis_name="s")

  @functools.partial(
      pl.kernel,
      mesh=mesh,
      out_type=jax.ShapeDtypeStruct((_B_TOTAL, _D), jnp.float32),
      scratch_types=[
          pltpu.VMEM((_B_PER_W,), jnp.int32),
          pltpu.VMEM((_B_PER_W, _D), jnp.float32),
          pltpu.SemaphoreType.DMA,
      ],
  )
  def gather_kernel(table_hbm, obs_hbm, cand_hbm, hid_hbm, out_hbm,
                    idx_v, rows_v, sem):
    wid = lax.axis_index("s") * _NC + lax.axis_index("c")
    base = wid * _B_PER_W

    @pl.when(wid < 5)
    def _():
      pltpu.sync_copy(obs_hbm.at[pl.ds(base, _B_PER_W)], idx_v)

    @pl.when(jnp.logical_and(wid >= 5, wid < 25))
    def _():
      pltpu.sync_copy(cand_hbm.at[pl.ds(base - _B_OBS, _B_PER_W)], idx_v)

    @pl.when(jnp.logical_and(wid >= 25, wid < _NACT))
    def _():
      pltpu.sync_copy(hid_hbm.at[pl.ds(base - _B_OBS - _B_CAND, _B_PER_W)],
                      idx_v)

    @pl.when(wid < _NACT)
    def _():
      copies = []
      for c in range(_NCHUNK):
        copies.append(
            pltpu.async_copy(
                table_hbm.at[idx_v.at[pl.ds(c * _CHUNK, _CHUNK)]],
                rows_v.at[pl.ds(c * _CHUNK, _CHUNK)],
                sem,
            ))
      for cp in copies:
        cp.wait()
      pltpu.sync_copy(rows_v, out_hbm.at[pl.ds(base, _B_PER_W)])

  return gather_kernel(table, obs_idx, cand_idx, hid_idx)


def _fused_body(w_ec_ref, b_ec_ref, b_el_ref, w_tc_ref, b_tc_ref, b_tl_ref,
                rows_ref, wl_e_ref, wl_t_ref, out_ref, m_scr):
  zcol = jnp.zeros((1, 1), jnp.float32)

  def wtilde(wl, w3_ref, c):
    shl = jnp.concatenate([wl[:, 1:], zcol], axis=1)
    shr = jnp.concatenate([zcol, wl[:, :-1]], axis=1)
    return (w3_ref[0, c, 0] * shl + w3_ref[0, c, 1] * wl
            + w3_ref[0, c, 2] * shr)

  wl_e = wl_e_ref[...]
  wl_t = wl_t_ref[...]
  w_t0 = wtilde(wl_t, w_tc_ref, 0).reshape(1, 1, _D)
  w_t1 = wtilde(wl_t, w_tc_ref, 1).reshape(1, 1, _D)
  w_e0 = wtilde(wl_e, w_ec_ref, 0).reshape(1, 1, _D)
  w_e1 = wtilde(wl_e, w_ec_ref, 1).reshape(1, 1, _D)
  c_t = b_tc_ref[0] * jnp.sum(wl_t) + b_tl_ref[0]
  c_e = b_ec_ref[0] * jnp.sum(wl_e) + b_el_ref[0]

  inv_n = jnp.float32(1.0 / _N)
  obs_mean = (jnp.sum(rows_ref[0:_B_OBS].reshape(_L, _N, _D), axis=1)
              * inv_n)
  hid_mean = (jnp.sum(rows_ref[_B_OBS + _B_CAND:_B_TOTAL]
                      .reshape(_L, _N, _D), axis=1) * inv_n)

  def mproj(mat, w):
    return jnp.sum(mat * w.reshape(1, _D), axis=1, keepdims=True)

  obs_e0 = mproj(obs_mean, w_e0)
  obs_e1 = mproj(obs_mean, w_e1)
  hid_t0 = mproj(hid_mean, w_t0)
  hid_t1 = mproj(hid_mean, w_t1)
  hid_e0 = mproj(hid_mean, w_e0)

  cand = rows_ref[_B_OBS:_B_OBS + _B_CAND].reshape(_L, _K, _D)

  a_mat = jnp.sum(cand * w_t0, axis=2)
  b_mat = jnp.sum(cand * w_t1, axis=2)
  e_mat = jnp.sum(cand * w_e1, axis=2)

  em = jax.nn.sigmoid(c_e + obs_e0 + e_mat)

  s_logit = c_t + a_mat[:-1][:, :, None] + b_mat[1:][:, None, :]
  m_scr[...] = jnp.exp(jax.nn.sigmoid(s_logit) + em[1:][:, None, :])

  z0 = jnp.exp(em[0:1, :])
  logs = []
  srcs = [m_scr[t] for t in range(_L - 1)]
  level = 0
  while len(srcs) > 1:
    level += 1
    rescale = level in (3, 6)
    dsts = []
    for k in range(len(srcs) // 2):
      p = jnp.dot(srcs[2 * k], srcs[2 * k + 1],
                  preferred_element_type=jnp.float32)
      if rescale:
        mx = jnp.max(p, axis=(0, 1), keepdims=True)
        logs.append(jnp.log(mx))
        p = p * (1.0 / mx)
      dsts.append(p)
    if len(srcs) % 2:
      carry = srcs[-1]
      if rescale:
        mx = jnp.max(carry, axis=(0, 1), keepdims=True)
        logs.append(jnp.log(mx))
        carry = carry * (1.0 / mx)
      dsts.append(carry)
    srcs = dsts
  z = jnp.dot(z0, srcs[0], preferred_element_type=jnp.float32)
  logz = logs[0]
  for lg in logs[1:]:
    logz = logz + lg
  fwd = logz + jnp.log(jnp.sum(z, axis=1, keepdims=True))

  e_terms = jax.nn.sigmoid(c_e + hid_e0 + obs_e1)
  t_terms = jax.nn.sigmoid(c_t + hid_t0[1:] + hid_t1[:-1])
  gold = (jnp.sum(e_terms, axis=0, keepdims=True)
          + jnp.sum(t_terms, axis=0, keepdims=True))

  out_ref[...] = fwd - gold


def _fused_score(w_ec, b_ec, b_el, w_tc, b_tc, b_tl, rows, wl_e, wl_t):
  return pl.pallas_call(
      _fused_body,
      out_shape=jax.ShapeDtypeStruct((1, 1), jnp.float32),
      in_specs=[
          pl.BlockSpec(memory_space=pltpu.SMEM),
          pl.BlockSpec(memory_space=pltpu.SMEM),
          pl.BlockSpec(memory_space=pltpu.SMEM),
          pl.BlockSpec(memory_space=pltpu.SMEM),
          pl.BlockSpec(memory_space=pltpu.SMEM),
          pl.BlockSpec(memory_space=pltpu.SMEM),
          pl.BlockSpec(memory_space=pltpu.VMEM),
          pl.BlockSpec(memory_space=pltpu.VMEM),
          pl.BlockSpec(memory_space=pltpu.VMEM),
      ],
      scratch_shapes=[pltpu.VMEM((_L - 1, _K, _K), jnp.float32)],
  )(w_ec, b_ec, b_el, w_tc, b_tc, b_tl, rows, wl_e, wl_t)


def kernel(W_embed, w_ec, b_ec, w_el, b_el, w_tc, b_tc, w_tl, b_tl,
           observed, candidates, hidden_states):
  rows = _sc_gather(W_embed,
                    observed.reshape(-1).astype(jnp.int32),
                    candidates.reshape(-1).astype(jnp.int32),
                    hidden_states.reshape(-1).astype(jnp.int32))
  out = _fused_score(w_ec, b_ec, b_el, w_tc, b_tc, b_tl, rows, w_el, w_tl)
  return out.reshape((1,))


# speedup vs baseline: 1.0316x; 1.0316x over previous
import functools

import jax
import jax.numpy as jnp
from jax import lax
from jax.experimental import pallas as pl
from jax.experimental.pallas import tpu as pltpu
from jax.experimental.pallas import tpu_sc as plsc

_V = 100000
_D = 128
_L = 200
_K = 64
_N = 16
_B_OBS = _L * _N
_B_CAND = _L * _K
_B_TOTAL = _B_OBS + _B_CAND + _B_OBS

_NC = 2
_NS = 16
_NW = _NC * _NS
_B_PER_W = _B_TOTAL // _NW
_CHUNK = 120
_NCHUNK = _B_PER_W // _CHUNK


def _sc_gather(table, idx):
  mesh = plsc.VectorSubcoreMesh(core_axis_name="c", subcore_axis_name="s")

  @functools.partial(
      pl.kernel,
      mesh=mesh,
      out_type=jax.ShapeDtypeStruct((_B_TOTAL, _D), jnp.float32),
      scratch_types=[
          pltpu.VMEM((_B_PER_W,), jnp.int32),
          pltpu.VMEM((_B_PER_W, _D), jnp.float32),
          pltpu.SemaphoreType.DMA,
      ],
  )
  def gather_kernel(table_hbm, idx_hbm, out_hbm, idx_v, rows_v, sem):
    wid = lax.axis_index("s") * _NC + lax.axis_index("c")
    base = wid * _B_PER_W
    pltpu.sync_copy(idx_hbm.at[pl.ds(base, _B_PER_W)], idx_v)
    copies = []
    for c in range(_NCHUNK):
      copies.append(
          pltpu.async_copy(
              table_hbm.at[idx_v.at[pl.ds(c * _CHUNK, _CHUNK)]],
              rows_v.at[pl.ds(c * _CHUNK, _CHUNK)],
              sem,
          ))
    for cp in copies:
      cp.wait()
    pltpu.sync_copy(rows_v, out_hbm.at[pl.ds(base, _B_PER_W)])

  return gather_kernel(table, idx)


def _fused_body(w_ec_ref, b_ec_ref, b_el_ref, w_tc_ref, b_tc_ref, b_tl_ref,
                rows_ref, wl_e_ref, wl_t_ref, out_ref, m_scr):
  zcol = jnp.zeros((1, 1), jnp.float32)

  def wtilde(wl, w3_ref, c):
    shl = jnp.concatenate([wl[:, 1:], zcol], axis=1)
    shr = jnp.concatenate([zcol, wl[:, :-1]], axis=1)
    return (w3_ref[0, c, 0] * shl + w3_ref[0, c, 1] * wl
            + w3_ref[0, c, 2] * shr)

  wl_e = wl_e_ref[...]
  wl_t = wl_t_ref[...]
  w_t0 = wtilde(wl_t, w_tc_ref, 0).reshape(1, 1, _D)
  w_t1 = wtilde(wl_t, w_tc_ref, 1).reshape(1, 1, _D)
  w_e0 = wtilde(wl_e, w_ec_ref, 0).reshape(1, 1, _D)
  w_e1 = wtilde(wl_e, w_ec_ref, 1).reshape(1, 1, _D)
  c_t = b_tc_ref[0] * jnp.sum(wl_t) + b_tl_ref[0]
  c_e = b_ec_ref[0] * jnp.sum(wl_e) + b_el_ref[0]

  inv_n = jnp.float32(1.0 / _N)
  obs_mean = (jnp.sum(rows_ref[0:_B_OBS].reshape(_L, _N, _D), axis=1)
              * inv_n)
  hid_mean = (jnp.sum(rows_ref[_B_OBS + _B_CAND:_B_TOTAL]
                      .reshape(_L, _N, _D), axis=1) * inv_n)

  def mproj(mat, w):
    return jnp.sum(mat * w.reshape(1, _D), axis=1, keepdims=True)

  obs_e0 = mproj(obs_mean, w_e0)
  obs_e1 = mproj(obs_mean, w_e1)
  hid_t0 = mproj(hid_mean, w_t0)
  hid_t1 = mproj(hid_mean, w_t1)
  hid_e0 = mproj(hid_mean, w_e0)

  cand = rows_ref[_B_OBS:_B_OBS + _B_CAND].reshape(_L, _K, _D)

  a_mat = jnp.sum(cand * w_t0, axis=2)
  b_mat = jnp.sum(cand * w_t1, axis=2)
  e_mat = jnp.sum(cand * w_e1, axis=2)

  em = jax.nn.sigmoid(c_e + obs_e0 + e_mat)

  s_logit = c_t + a_mat[:-1][:, :, None] + b_mat[1:][:, None, :]
  m_scr[...] = jnp.exp(jax.nn.sigmoid(s_logit) + em[1:][:, None, :])

  z0 = jnp.exp(em[0:1, :])
  logs = []
  srcs = [m_scr[t] for t in range(_L - 1)]
  level = 0
  while len(srcs) > 1:
    level += 1
    rescale = level in (3, 6)
    dsts = []
    for k in range(len(srcs) // 2):
      p = jnp.dot(srcs[2 * k], srcs[2 * k + 1],
                  preferred_element_type=jnp.float32)
      if rescale:
        mx = jnp.max(p, axis=(0, 1), keepdims=True)
        logs.append(jnp.log(mx))
        p = p * (1.0 / mx)
      dsts.append(p)
    if len(srcs) % 2:
      carry = srcs[-1]
      if rescale:
        mx = jnp.max(carry, axis=(0, 1), keepdims=True)
        logs.append(jnp.log(mx))
        carry = carry * (1.0 / mx)
      dsts.append(carry)
    srcs = dsts
  z = jnp.dot(z0, srcs[0], preferred_element_type=jnp.float32)
  logz = logs[0]
  for lg in logs[1:]:
    logz = logz + lg
  fwd = logz + jnp.log(jnp.sum(z, axis=1, keepdims=True))

  e_terms = jax.nn.sigmoid(c_e + hid_e0 + obs_e1)
  t_terms = jax.nn.sigmoid(c_t + hid_t0[1:] + hid_t1[:-1])
  gold = (jnp.sum(e_terms, axis=0, keepdims=True)
          + jnp.sum(t_terms, axis=0, keepdims=True))

  out_ref[...] = fwd - gold


def _fused_score(w_ec, b_ec, b_el, w_tc, b_tc, b_tl, rows, wl_e, wl_t):
  return pl.pallas_call(
      _fused_body,
      out_shape=jax.ShapeDtypeStruct((1, 1), jnp.float32),
      in_specs=[
          pl.BlockSpec(memory_space=pltpu.SMEM),
          pl.BlockSpec(memory_space=pltpu.SMEM),
          pl.BlockSpec(memory_space=pltpu.SMEM),
          pl.BlockSpec(memory_space=pltpu.SMEM),
          pl.BlockSpec(memory_space=pltpu.SMEM),
          pl.BlockSpec(memory_space=pltpu.SMEM),
          pl.BlockSpec(memory_space=pltpu.VMEM),
          pl.BlockSpec(memory_space=pltpu.VMEM),
          pl.BlockSpec(memory_space=pltpu.VMEM),
      ],
      scratch_shapes=[pltpu.VMEM((_L - 1, _K, _K), jnp.float32)],
  )(w_ec, b_ec, b_el, w_tc, b_tc, b_tl, rows, wl_e, wl_t)


def kernel(W_embed, w_ec, b_ec, w_el, b_el, w_tc, b_tc, w_tl, b_tl,
           observed, candidates, hidden_states):
  idx = jnp.concatenate([
      observed.reshape(-1),
      candidates.reshape(-1),
      hidden_states.reshape(-1),
  ]).astype(jnp.int32)
  rows = _sc_gather(W_embed, idx)
  out = _fused_score(w_ec, b_ec, b_el, w_tc, b_tc, b_tl, rows, w_el, w_tl)
  return out.reshape((1,))
